# Initial kernel scaffold; baseline (speedup 1.0000x reference)
import jax
import jax.numpy as jnp
from jax.experimental import pallas as pl


def kernel(params, node_features, edge_list, edge_features, timestamps, target_time):
    raise NotImplementedError("write your pallas kernel here")


# speedup vs baseline: 1.5377x; 1.5377x over previous
import functools
import math

import jax
import jax.numpy as jnp
import numpy as np
from jax import lax
from jax.experimental import pallas as pl
from jax.experimental.pallas import tpu as pltpu
from jax.experimental.pallas import tpu_sc as plsc

HID = 64
NFD = 128
EFD = 16
N = 10000
L = 2
E = 160000
NR = 10240
NRTOT = 2 * NR
EPAD = 163840
EF = 2 * EPAD
CH = 128
NWORK = 32
EPW = EPAD // 16
NCHUNK = EPW // CH
ROWS_PER_TILE = NR // 16
AW = 80


def _pos_encoding(Lx, d):
    pos = np.arange(Lx)[:, None].astype(np.float32)
    div = np.exp(np.arange(0, d, 2).astype(np.float32) * -(math.log(10000.0) / d))
    pe = np.zeros((Lx, d), np.float32)
    pe[:, 0::2] = np.sin(pos * div)
    pe[:, 1::2] = np.cos(pos * div)
    return pe


_PE = _pos_encoding(L, HID)


def _whole(shape):
    return pl.BlockSpec(shape, lambda *_: tuple(0 for _ in shape))


def _mlp2_body(x, W1, b1, W2, b2):
    hcur = jnp.maximum(jnp.dot(x, W1, preferred_element_type=jnp.float32) + b1, 0.0)
    return jnp.dot(hcur, W2, preferred_element_type=jnp.float32) + b2


def _node_enc_k(nf_ref, W1_ref, b1_ref, W2_ref, b2_ref, out_ref):
    out_ref[...] = _mlp2_body(nf_ref[...], W1_ref[...], b1_ref[...],
                              W2_ref[...], b2_ref[...])


def _node_enc(nf_f, W1, b1, W2, b2):
    B = 2048
    grid = (NRTOT // B,)
    return pl.pallas_call(
        _node_enc_k,
        grid=grid,
        in_specs=[pl.BlockSpec((B, NFD), lambda i: (i, 0)),
                  _whole(W1.shape), _whole(b1.shape),
                  _whole(W2.shape), _whole(b2.shape)],
        out_specs=pl.BlockSpec((B, HID), lambda i: (i, 0)),
        out_shape=jax.ShapeDtypeStruct((NRTOT, HID), jnp.float32),
    )(nf_f, W1, b1, W2, b2)


def _edge_enc_k(ef_ref, W1_ref, b1_ref, W2_ref, b2_ref,
                Wb0_ref, bb0_ref, Wb1_ref, bb1_ref, e0_ref, e1_ref):
    e = _mlp2_body(ef_ref[...], W1_ref[...], b1_ref[...], W2_ref[...], b2_ref[...])
    e0_ref[...] = jnp.dot(e, Wb0_ref[...], preferred_element_type=jnp.float32) + bb0_ref[...]
    e1_ref[...] = jnp.dot(e, Wb1_ref[...], preferred_element_type=jnp.float32) + bb1_ref[...]


def _edge_enc(ef_f, W1, b1, W2, b2, Wb0, bb0, Wb1, bb1):
    B = 4096
    grid = (EF // B,)
    return pl.pallas_call(
        _edge_enc_k,
        grid=grid,
        in_specs=[pl.BlockSpec((B, EFD), lambda i: (i, 0)),
                  _whole(W1.shape), _whole(b1.shape),
                  _whole(W2.shape), _whole(b2.shape),
                  _whole(Wb0.shape), _whole(bb0.shape),
                  _whole(Wb1.shape), _whole(bb1.shape)],
        out_specs=[pl.BlockSpec((B, HID), lambda i: (i, 0)),
                   pl.BlockSpec((B, HID), lambda i: (i, 0))],
        out_shape=[jax.ShapeDtypeStruct((EF, HID), jnp.float32),
                   jax.ShapeDtypeStruct((EF, HID), jnp.float32)],
    )(ef_f, W1, b1, W2, b2, Wb0, bb0, Wb1, bb1)


def _tbuild_k(h_ref, Wa_ref, Wc_ref, ga_ref, gc_ref):
    h = h_ref[...]
    ga_ref[...] = jnp.dot(h, Wa_ref[...], preferred_element_type=jnp.float32)
    gc_ref[...] = jnp.dot(h, Wc_ref[...], preferred_element_type=jnp.float32)


def _tbuild(h_f, Wa, Wc):
    B = 2048
    grid = (NRTOT // B,)
    return pl.pallas_call(
        _tbuild_k,
        grid=grid,
        in_specs=[pl.BlockSpec((B, HID), lambda i: (i, 0)),
                  _whole(Wa.shape), _whole(Wc.shape)],
        out_specs=[pl.BlockSpec((B, HID), lambda i: (i, 0)),
                   pl.BlockSpec((B, HID), lambda i: (i, 0))],
        out_shape=[jax.ShapeDtypeStruct((NRTOT, HID), jnp.float32),
                   jax.ShapeDtypeStruct((NRTOT, HID), jnp.float32)],
    )(h_f, Wa, Wc)


def _combine_k(h_ref, A_ref, W2a_ref, Wg1_ref, bg1_ref, Wg2_ref, bg2_ref, out_ref):
    msg = jnp.dot(A_ref[...], W2a_ref[...], preferred_element_type=jnp.float32)
    new_h = h_ref[...] + msg
    out_ref[...] = _mlp2_body(new_h, Wg1_ref[...], bg1_ref[...],
                              Wg2_ref[...], bg2_ref[...])


def _combine(h_f, A, W2aug, Wg1, bg1, Wg2, bg2):
    B = 2048
    grid = (NRTOT // B,)
    return pl.pallas_call(
        _combine_k,
        grid=grid,
        in_specs=[pl.BlockSpec((B, HID), lambda i: (i, 0)),
                  pl.BlockSpec((B, AW), lambda i: (i, 0)),
                  _whole(W2aug.shape),
                  _whole(Wg1.shape), _whole(bg1.shape),
                  _whole(Wg2.shape), _whole(bg2.shape)],
        out_specs=pl.BlockSpec((B, HID), lambda i: (i, 0)),
        out_shape=jax.ShapeDtypeStruct((NRTOT, HID), jnp.float32),
    )(h_f, A, W2aug, Wg1, bg1, Wg2, bg2)


def _ln(x, g, b):
    m = jnp.mean(x, axis=-1, keepdims=True)
    v = jnp.mean((x - m) ** 2, axis=-1, keepdims=True)
    return (x - m) / jnp.sqrt(v + 1e-5) * g + b


def _tail_k(h_ref, ts_ref, tt_ref, pe_ref,
            ttW1_ref, ttb1_ref, ttW2_ref, ttb2_ref,
            gtW1_ref, gtb1_ref, gtW2_ref, gtb2_ref,
            Wqkv_ref, bqkv_ref, Wo_ref, bo_ref,
            Wf1_ref, bf1_ref, Wf2_ref, bf2_ref,
            g1_ref, be1_ref, g2_ref, be2_ref,
            cW1_ref, cb1_ref, cW2_ref, cb2_ref,
            sW1_ref, sb1_ref, sW2_ref, sb2_ref,
            ctx_ref, logits_ref):
    s0 = jnp.mean(h_ref[0:N, :], axis=0)
    s1 = jnp.mean(h_ref[NR:NR + N, :], axis=0)
    ts_emb = jnp.concatenate([s0[None, :], s1[None, :]], axis=0)
    time_emb = _mlp2_body_bc(ts_ref[...], ttW1_ref[...], ttb1_ref[...],
                             ttW2_ref[...], ttb2_ref[...])
    x = ts_emb + time_emb + pe_ref[...]
    inv = 1.0 / math.sqrt(HID // 8)
    for l in range(6):
        qkv = jnp.dot(x, Wqkv_ref[l], preferred_element_type=jnp.float32) + bqkv_ref[l]
        q = qkv[:, 0:HID]
        k = qkv[:, HID:2 * HID]
        v = qkv[:, 2 * HID:3 * HID]
        outs = []
        for hh in range(8):
            sl = slice(hh * 8, hh * 8 + 8)
            qh, kh, vh = q[:, sl], k[:, sl], v[:, sl]
            s00 = jnp.sum(qh[0] * kh[0]) * inv
            s01 = jnp.sum(qh[0] * kh[1]) * inv
            s10 = jnp.sum(qh[1] * kh[0]) * inv
            s11 = jnp.sum(qh[1] * kh[1]) * inv
            m0 = jnp.maximum(s00, s01)
            e00 = jnp.exp(s00 - m0)
            e01 = jnp.exp(s01 - m0)
            a00 = e00 / (e00 + e01)
            a01 = e01 / (e00 + e01)
            m1 = jnp.maximum(s10, s11)
            e10 = jnp.exp(s10 - m1)
            e11 = jnp.exp(s11 - m1)
            a10 = e10 / (e10 + e11)
            a11 = e11 / (e10 + e11)
            o0 = a00 * vh[0] + a01 * vh[1]
            o1 = a10 * vh[0] + a11 * vh[1]
            outs.append(jnp.concatenate([o0[None, :], o1[None, :]], axis=0))
        o = jnp.concatenate(outs, axis=1)
        o = jnp.dot(o, Wo_ref[l], preferred_element_type=jnp.float32) + bo_ref[l]
        x = _ln(x + o, g1_ref[l], be1_ref[l])
        ff = jnp.maximum(jnp.dot(x, Wf1_ref[l], preferred_element_type=jnp.float32) + bf1_ref[l], 0.0)
        ff = jnp.dot(ff, Wf2_ref[l], preferred_element_type=jnp.float32) + bf2_ref[l]
        x = _ln(x + ff, g2_ref[l], be2_ref[l])
    seq_ctx = x[1]
    t_ctx = _mlp2_body_bc(tt_ref[...], gtW1_ref[...], gtb1_ref[...],
                          gtW2_ref[...], gtb2_ref[...])
    ctx_in = jnp.concatenate([s1[None, :], seq_ctx[None, :], t_ctx], axis=1)
    ctx = _mlp2_body(ctx_in, cW1_ref[...], cb1_ref[...], cW2_ref[...], cb2_ref[...])
    ctx_ref[...] = ctx
    logits_ref[...] = _mlp2_body(ctx, sW1_ref[...], sb1_ref[...],
                                 sW2_ref[...], sb2_ref[...])


def _mlp2_body_bc(x1, W1row, b1, W2, b2):
    hcur = jnp.maximum(x1 * W1row + b1, 0.0)
    return jnp.dot(hcur, W2, preferred_element_type=jnp.float32) + b2


def _tail(h_f, ts, tt, args):
    in_specs = [_whole(h_f.shape), _whole(ts.shape), _whole(tt.shape)]
    ops = [h_f, ts, tt]
    for a in args:
        in_specs.append(_whole(a.shape))
        ops.append(a)
    return pl.pallas_call(
        _tail_k,
        in_specs=in_specs,
        out_specs=[pl.BlockSpec((1, HID), lambda: (0, 0)),
                   pl.BlockSpec((1, 10000), lambda: (0, 0))],
        out_shape=[jax.ShapeDtypeStruct((1, HID), jnp.float32),
                   jax.ShapeDtypeStruct((1, 10000), jnp.float32)],
    )(*ops)


def _dst_k(x_ref, W1_ref, b1_ref, W2_ref, b2_ref, out_ref):
    out_ref[...] = _mlp2_body(x_ref[...], W1_ref[...], b1_ref[...],
                              W2_ref[...], b2_ref[...])


def _dst_mlp(x, W1, b1, W2, b2):
    return pl.pallas_call(
        _dst_k,
        in_specs=[_whole(x.shape), _whole(W1.shape), _whole(b1.shape),
                  _whole(W2.shape), _whole(b2.shape)],
        out_specs=pl.BlockSpec((1, 10000), lambda: (0, 0)),
        out_shape=jax.ShapeDtypeStruct((1, 10000), jnp.float32),
    )(x, W1, b1, W2, b2)


def _sc_msgpass_body(T_hbm, srcf_hbm, dstf_hbm, ee_hbm, out_hbm,
                     src_v, dst_v, srcg_v, dstg_v, ee_v,
                     ts_a, ts_b, td_a, td_b, msg_v, A_sh):
    cid = lax.axis_index("c")
    sid = lax.axis_index("s")

    zero16 = jnp.zeros((16,), jnp.float32)

    def _zrow(r, carry):
        for j in range(AW // 16):
            msg_v[r, pl.ds(j * 16, 16)] = zero16
        return carry
    lax.fori_loop(0, CH, _zrow, 0)

    def _zcp(kk, carry):
        pltpu.sync_copy(msg_v, A_sh.at[pl.ds(sid * ROWS_PER_TILE + kk * CH, CH)])
        return carry
    lax.fori_loop(0, ROWS_PER_TILE // CH, _zcp, 0)

    iot = lax.iota(jnp.int32, 16)
    one0 = jnp.where(iot == 0, jnp.float32(1.0), jnp.float32(0.0))

    def _ones(r, carry):
        msg_v[r, pl.ds(HID, 16)] = one0
        return carry
    lax.fori_loop(0, CH, _ones, 0)

    plsc.subcore_barrier()

    base0 = cid * EPAD + sid * EPW
    goff = cid * NR

    def _chunk(ci, carry):
        base = base0 + ci * CH
        pltpu.sync_copy(srcf_hbm.at[pl.ds(base, CH)], src_v)
        pltpu.sync_copy(dstf_hbm.at[pl.ds(base, CH)], dst_v)
        pltpu.sync_copy(ee_hbm.at[pl.ds(base, CH)], ee_v)

        def _adj(j, c2):
            sl = pl.ds(j * 16, 16)
            srcg_v[sl] = src_v[sl] + goff
            dstg_v[sl] = dst_v[sl] + goff
            return c2
        lax.fori_loop(0, CH // 16, _adj, 0)

        pltpu.sync_copy(T_hbm.at[srcg_v], ts_a)
        pltpu.sync_copy(T_hbm.at[dstg_v], td_a)

        def _adj2(j, c2):
            sl = pl.ds(j * 16, 16)
            srcg_v[sl] = srcg_v[sl] + NRTOT
            dstg_v[sl] = dstg_v[sl] + NRTOT
            return c2
        lax.fori_loop(0, CH // 16, _adj2, 0)

        pltpu.sync_copy(T_hbm.at[srcg_v], ts_b)
        pltpu.sync_copy(T_hbm.at[dstg_v], td_b)

        def _row_d(r, c2):
            for j in range(HID // 16):
                sl = pl.ds(j * 16, 16)
                msg_v[r, sl] = jnp.maximum(ts_a[r, sl] + ee_v[r, sl] + td_b[r, sl], 0.0)
            return c2
        lax.fori_loop(0, CH, _row_d, 0)
        pltpu.sync_copy(msg_v, A_sh.at[dst_v], add=True)

        def _row_s(r, c2):
            for j in range(HID // 16):
                sl = pl.ds(j * 16, 16)
                msg_v[r, sl] = jnp.maximum(td_a[r, sl] + ee_v[r, sl] + ts_b[r, sl], 0.0)
            return c2
        lax.fori_loop(0, CH, _row_s, 0)
        pltpu.sync_copy(msg_v, A_sh.at[src_v], add=True)
        return carry
    lax.fori_loop(0, NCHUNK, _chunk, 0)

    plsc.subcore_barrier()

    def _wb(kk, carry):
        r0 = sid * ROWS_PER_TILE + kk * CH
        pltpu.sync_copy(A_sh.at[pl.ds(r0, CH)], msg_v)
        pltpu.sync_copy(msg_v, out_hbm.at[pl.ds(goff + r0, CH)])
        return carry
    lax.fori_loop(0, ROWS_PER_TILE // CH, _wb, 0)


@functools.cache
def _get_sc_msgpass():
    mesh = plsc.VectorSubcoreMesh(core_axis_name="c", subcore_axis_name="s",
                                  num_cores=2, num_subcores=16)
    return pl.kernel(
        _sc_msgpass_body,
        mesh=mesh,
        compiler_params=pltpu.CompilerParams(use_tc_tiling_on_sc=False),
        out_type=jax.ShapeDtypeStruct((NRTOT, AW), jnp.float32),
        scratch_types=[
            pltpu.VMEM((CH,), jnp.int32),
            pltpu.VMEM((CH,), jnp.int32),
            pltpu.VMEM((CH,), jnp.int32),
            pltpu.VMEM((CH,), jnp.int32),
            pltpu.VMEM((CH, HID), jnp.float32),
            pltpu.VMEM((CH, HID), jnp.float32),
            pltpu.VMEM((CH, HID), jnp.float32),
            pltpu.VMEM((CH, HID), jnp.float32),
            pltpu.VMEM((CH, HID), jnp.float32),
            pltpu.VMEM((CH, AW), jnp.float32),
            pltpu.VMEM_SHARED((NR, AW), jnp.float32),
        ],
    )


def _sc_debug_emu(T, src_f, dst_f, Ee):
    As = []
    for sct in range(2):
        sl = slice(sct * EPAD, (sct + 1) * EPAD)
        srcl, dstl, ee = src_f[sl], dst_f[sl], Ee[sl]
        srcg, dstg = srcl + sct * NR, dstl + sct * NR
        rd = jax.nn.relu(T[srcg] + ee + T[dstg + NRTOT])
        rs = jax.nn.relu(T[dstg] + ee + T[srcg + NRTOT])
        ones = jnp.ones((EPAD, 1), jnp.float32)
        zpad = jnp.zeros((EPAD, AW - HID - 1), jnp.float32)
        rowd = jnp.concatenate([rd, ones, zpad], 1)
        rows = jnp.concatenate([rs, ones, zpad], 1)
        Acc = jnp.zeros((NR, AW), jnp.float32)
        Acc = Acc.at[dstl].add(rowd).at[srcl].add(rows)
        As.append(Acc)
    return jnp.concatenate(As, axis=0)


def kernel(params, node_features, edge_list, edge_features, timestamps, target_time):
    f32 = jnp.float32

    def r2(b):
        return b.reshape(1, -1)

    (nW1, nb1), (nW2, nb2) = params['node_enc']
    (eW1, eb1), (eW2, eb2) = params['edge_enc']
    (gW1, gb1), (gW2, gb2) = params['agg']
    msg = params['msg']
    W1a, W1c, Wb, bb, W2aug = [], [], [], [], []
    for lp in msg:
        (W1, b1), (W2, b2) = lp
        W1a.append(W1[0:HID])
        Wb.append(W1[HID:2 * HID])
        W1c.append(W1[2 * HID:3 * HID])
        bb.append(r2(b1))
        w2a = jnp.concatenate([W2, b2[None, :]], axis=0)
        W2aug.append(jnp.pad(w2a, ((0, AW - HID - 1), (0, 0))))

    nf_f = jnp.pad(node_features, ((0, 0), (0, NR - N), (0, 0))).reshape(NRTOT, NFD)
    src = edge_list[:, :, 0]
    dst = edge_list[:, :, 1]
    src_f = jnp.pad(src, ((0, 0), (0, EPAD - E)), constant_values=N).reshape(EF)
    dst_f = jnp.pad(dst, ((0, 0), (0, EPAD - E)), constant_values=N).reshape(EF)
    ef_f = jnp.pad(edge_features, ((0, 0), (0, EPAD - E), (0, 0))).reshape(EF, EFD)

    h_f = _node_enc(nf_f, nW1, r2(nb1), nW2, r2(nb2))
    Ee0, Ee1 = _edge_enc(ef_f, eW1, r2(eb1), eW2, r2(eb2),
                         Wb[0], bb[0], Wb[1], bb[1])
    Ee = (Ee0, Ee1)

    for li in range(2):
        G1, G3 = _tbuild(h_f, W1a[li], W1c[li])
        T = jnp.concatenate([G1, G3], axis=0)
        A = _get_sc_msgpass()(T, src_f, dst_f, Ee[li])
        h_f = _combine(h_f, A, W2aug[li], gW1, r2(gb1), gW2, r2(gb2))

    (ttW1, ttb1), (ttW2, ttb2) = params['t_time']
    (gtW1, gtb1), (gtW2, gtb2) = params['g_time']
    tf = params['tf']
    stk = lambda key: jnp.stack([lp[key] for lp in tf])
    stk2 = lambda key: jnp.stack([lp[key][None, :] for lp in tf])
    targs = [jnp.asarray(_PE),
             ttW1, r2(ttb1), ttW2, r2(ttb2),
             gtW1, r2(gtb1), gtW2, r2(gtb2),
             stk('Wqkv'), stk2('bqkv'), stk('Wo'), stk2('bo'),
             stk('W1'), stk2('b1'), stk('W2'), stk2('b2'),
             stk2('g1'), stk2('be1'), stk2('g2'), stk2('be2')]
    (cW1, cb1), (cW2, cb2) = params['ctx']
    (sW1, sb1), (sW2, sb2) = params['srcp']
    targs += [cW1, r2(cb1), cW2, r2(cb2), sW1, r2(sb1), sW2, r2(sb2)]
    ts2 = timestamps[:, None].astype(f32)
    tt2 = target_time.reshape(1, 1).astype(f32)
    ctx, src_logits2 = _tail(h_f, ts2, tt2, targs)
    src_logits = src_logits2[0]

    src_node = jax.random.categorical(jax.random.key(42), src_logits)
    src_emb = jax.lax.dynamic_slice(h_f, (NR + src_node, 0), (1, HID))
    (dW1, db1), (dW2, db2) = params['dstp']
    x_d = jnp.concatenate([ctx, src_emb], axis=1)
    dst_logits = _dst_mlp(x_d, dW1, r2(db1), dW2, r2(db2))[0]

    return src_logits, dst_logits, src_node


# speedup vs baseline: 1.8595x; 1.2092x over previous
import functools
import math

import jax
import jax.numpy as jnp
import numpy as np
from jax import lax
from jax.experimental import pallas as pl
from jax.experimental.pallas import tpu as pltpu
from jax.experimental.pallas import tpu_sc as plsc

HID = 64
NFD = 128
EFD = 16
N = 10000
L = 2
E = 160000
NR = 10240
NRTOT = 2 * NR
EPAD = 163840
EF = 2 * EPAD
CH = 128
NWORK = 32
EPW = EPAD // 16
NCHUNK = EPW // CH
ROWS_PER_TILE = NR // 16
AW = 80


def _pos_encoding(Lx, d):
    pos = np.arange(Lx)[:, None].astype(np.float32)
    div = np.exp(np.arange(0, d, 2).astype(np.float32) * -(math.log(10000.0) / d))
    pe = np.zeros((Lx, d), np.float32)
    pe[:, 0::2] = np.sin(pos * div)
    pe[:, 1::2] = np.cos(pos * div)
    return pe


_PE = _pos_encoding(L, HID)


def _whole(shape):
    return pl.BlockSpec(shape, lambda *_: tuple(0 for _ in shape))


def _mlp2_body(x, W1, b1, W2, b2):
    hcur = jnp.maximum(jnp.dot(x, W1, preferred_element_type=jnp.float32) + b1, 0.0)
    return jnp.dot(hcur, W2, preferred_element_type=jnp.float32) + b2


def _node_enc_k(nf_ref, W1_ref, b1_ref, W2_ref, b2_ref, out_ref):
    out_ref[...] = _mlp2_body(nf_ref[...], W1_ref[...], b1_ref[...],
                              W2_ref[...], b2_ref[...])


def _node_enc(nf_f, W1, b1, W2, b2):
    B = 2048
    grid = (NRTOT // B,)
    return pl.pallas_call(
        _node_enc_k,
        grid=grid,
        in_specs=[pl.BlockSpec((B, NFD), lambda i: (i, 0)),
                  _whole(W1.shape), _whole(b1.shape),
                  _whole(W2.shape), _whole(b2.shape)],
        out_specs=pl.BlockSpec((B, HID), lambda i: (i, 0)),
        out_shape=jax.ShapeDtypeStruct((NRTOT, HID), jnp.float32),
    )(nf_f, W1, b1, W2, b2)


def _edge_enc_k(ef_ref, W1_ref, b1_ref, W2_ref, b2_ref,
                Wb0_ref, bb0_ref, Wb1_ref, bb1_ref, e0_ref, e1_ref):
    e = _mlp2_body(ef_ref[...], W1_ref[...], b1_ref[...], W2_ref[...], b2_ref[...])
    e0_ref[...] = jnp.dot(e, Wb0_ref[...], preferred_element_type=jnp.float32) + bb0_ref[...]
    e1_ref[...] = jnp.dot(e, Wb1_ref[...], preferred_element_type=jnp.float32) + bb1_ref[...]


def _edge_enc(ef_f, W1, b1, W2, b2, Wb0, bb0, Wb1, bb1):
    B = 4096
    grid = (EF // B,)
    return pl.pallas_call(
        _edge_enc_k,
        grid=grid,
        in_specs=[pl.BlockSpec((B, EFD), lambda i: (i, 0)),
                  _whole(W1.shape), _whole(b1.shape),
                  _whole(W2.shape), _whole(b2.shape),
                  _whole(Wb0.shape), _whole(bb0.shape),
                  _whole(Wb1.shape), _whole(bb1.shape)],
        out_specs=[pl.BlockSpec((B, HID), lambda i: (i, 0)),
                   pl.BlockSpec((B, HID), lambda i: (i, 0))],
        out_shape=[jax.ShapeDtypeStruct((EF, HID), jnp.float32),
                   jax.ShapeDtypeStruct((EF, HID), jnp.float32)],
    )(ef_f, W1, b1, W2, b2, Wb0, bb0, Wb1, bb1)


def _tbuild_k(h_ref, Wa_ref, Wc_ref, ga_ref, gc_ref):
    h = h_ref[...]
    ga_ref[...] = jnp.dot(h, Wa_ref[...], preferred_element_type=jnp.float32)
    gc_ref[...] = jnp.dot(h, Wc_ref[...], preferred_element_type=jnp.float32)


def _tbuild(h_f, Wa, Wc):
    B = 2048
    grid = (NRTOT // B,)
    return pl.pallas_call(
        _tbuild_k,
        grid=grid,
        in_specs=[pl.BlockSpec((B, HID), lambda i: (i, 0)),
                  _whole(Wa.shape), _whole(Wc.shape)],
        out_specs=[pl.BlockSpec((B, HID), lambda i: (i, 0)),
                   pl.BlockSpec((B, HID), lambda i: (i, 0))],
        out_shape=[jax.ShapeDtypeStruct((NRTOT, HID), jnp.float32),
                   jax.ShapeDtypeStruct((NRTOT, HID), jnp.float32)],
    )(h_f, Wa, Wc)


def _combine_k(h_ref, A_ref, W2a_ref, Wg1_ref, bg1_ref, Wg2_ref, bg2_ref, out_ref):
    msg = jnp.dot(A_ref[...], W2a_ref[...], preferred_element_type=jnp.float32)
    new_h = h_ref[...] + msg
    out_ref[...] = _mlp2_body(new_h, Wg1_ref[...], bg1_ref[...],
                              Wg2_ref[...], bg2_ref[...])


def _combine(h_f, A, W2aug, Wg1, bg1, Wg2, bg2):
    B = 2048
    grid = (NRTOT // B,)
    return pl.pallas_call(
        _combine_k,
        grid=grid,
        in_specs=[pl.BlockSpec((B, HID), lambda i: (i, 0)),
                  pl.BlockSpec((B, AW), lambda i: (i, 0)),
                  _whole(W2aug.shape),
                  _whole(Wg1.shape), _whole(bg1.shape),
                  _whole(Wg2.shape), _whole(bg2.shape)],
        out_specs=pl.BlockSpec((B, HID), lambda i: (i, 0)),
        out_shape=jax.ShapeDtypeStruct((NRTOT, HID), jnp.float32),
    )(h_f, A, W2aug, Wg1, bg1, Wg2, bg2)


def _ln(x, g, b):
    m = jnp.mean(x, axis=-1, keepdims=True)
    v = jnp.mean((x - m) ** 2, axis=-1, keepdims=True)
    return (x - m) / jnp.sqrt(v + 1e-5) * g + b


def _tail_k(h_ref, ts_ref, tt_ref, pe_ref,
            ttW1_ref, ttb1_ref, ttW2_ref, ttb2_ref,
            gtW1_ref, gtb1_ref, gtW2_ref, gtb2_ref,
            Wqkv_ref, bqkv_ref, Wo_ref, bo_ref,
            Wf1_ref, bf1_ref, Wf2_ref, bf2_ref,
            g1_ref, be1_ref, g2_ref, be2_ref,
            cW1_ref, cb1_ref, cW2_ref, cb2_ref,
            sW1_ref, sb1_ref, sW2_ref, sb2_ref,
            ctx_ref, logits_ref):
    s0 = jnp.mean(h_ref[0:N, :], axis=0)
    s1 = jnp.mean(h_ref[NR:NR + N, :], axis=0)
    ts_emb = jnp.concatenate([s0[None, :], s1[None, :]], axis=0)
    time_emb = _mlp2_body_bc(ts_ref[...], ttW1_ref[...], ttb1_ref[...],
                             ttW2_ref[...], ttb2_ref[...])
    x = ts_emb + time_emb + pe_ref[...]
    inv = 1.0 / math.sqrt(HID // 8)
    for l in range(6):
        qkv = jnp.dot(x, Wqkv_ref[l], preferred_element_type=jnp.float32) + bqkv_ref[l]
        q = qkv[:, 0:HID]
        k = qkv[:, HID:2 * HID]
        v = qkv[:, 2 * HID:3 * HID]
        outs = []
        for hh in range(8):
            sl = slice(hh * 8, hh * 8 + 8)
            qh, kh, vh = q[:, sl], k[:, sl], v[:, sl]
            s00 = jnp.sum(qh[0] * kh[0]) * inv
            s01 = jnp.sum(qh[0] * kh[1]) * inv
            s10 = jnp.sum(qh[1] * kh[0]) * inv
            s11 = jnp.sum(qh[1] * kh[1]) * inv
            m0 = jnp.maximum(s00, s01)
            e00 = jnp.exp(s00 - m0)
            e01 = jnp.exp(s01 - m0)
            a00 = e00 / (e00 + e01)
            a01 = e01 / (e00 + e01)
            m1 = jnp.maximum(s10, s11)
            e10 = jnp.exp(s10 - m1)
            e11 = jnp.exp(s11 - m1)
            a10 = e10 / (e10 + e11)
            a11 = e11 / (e10 + e11)
            o0 = a00 * vh[0] + a01 * vh[1]
            o1 = a10 * vh[0] + a11 * vh[1]
            outs.append(jnp.concatenate([o0[None, :], o1[None, :]], axis=0))
        o = jnp.concatenate(outs, axis=1)
        o = jnp.dot(o, Wo_ref[l], preferred_element_type=jnp.float32) + bo_ref[l]
        x = _ln(x + o, g1_ref[l], be1_ref[l])
        ff = jnp.maximum(jnp.dot(x, Wf1_ref[l], preferred_element_type=jnp.float32) + bf1_ref[l], 0.0)
        ff = jnp.dot(ff, Wf2_ref[l], preferred_element_type=jnp.float32) + bf2_ref[l]
        x = _ln(x + ff, g2_ref[l], be2_ref[l])
    seq_ctx = x[1]
    t_ctx = _mlp2_body_bc(tt_ref[...], gtW1_ref[...], gtb1_ref[...],
                          gtW2_ref[...], gtb2_ref[...])
    ctx_in = jnp.concatenate([s1[None, :], seq_ctx[None, :], t_ctx], axis=1)
    ctx = _mlp2_body(ctx_in, cW1_ref[...], cb1_ref[...], cW2_ref[...], cb2_ref[...])
    ctx_ref[...] = ctx
    logits_ref[...] = _mlp2_body(ctx, sW1_ref[...], sb1_ref[...],
                                 sW2_ref[...], sb2_ref[...])


def _mlp2_body_bc(x1, W1row, b1, W2, b2):
    hcur = jnp.maximum(x1 * W1row + b1, 0.0)
    return jnp.dot(hcur, W2, preferred_element_type=jnp.float32) + b2


def _tail(h_f, ts, tt, args):
    in_specs = [_whole(h_f.shape), _whole(ts.shape), _whole(tt.shape)]
    ops = [h_f, ts, tt]
    for a in args:
        in_specs.append(_whole(a.shape))
        ops.append(a)
    return pl.pallas_call(
        _tail_k,
        in_specs=in_specs,
        out_specs=[pl.BlockSpec((1, HID), lambda: (0, 0)),
                   pl.BlockSpec((1, 10000), lambda: (0, 0))],
        out_shape=[jax.ShapeDtypeStruct((1, HID), jnp.float32),
                   jax.ShapeDtypeStruct((1, 10000), jnp.float32)],
    )(*ops)


def _dst_k(x_ref, W1_ref, b1_ref, W2_ref, b2_ref, out_ref):
    out_ref[...] = _mlp2_body(x_ref[...], W1_ref[...], b1_ref[...],
                              W2_ref[...], b2_ref[...])


def _dst_mlp(x, W1, b1, W2, b2):
    return pl.pallas_call(
        _dst_k,
        in_specs=[_whole(x.shape), _whole(W1.shape), _whole(b1.shape),
                  _whole(W2.shape), _whole(b2.shape)],
        out_specs=pl.BlockSpec((1, 10000), lambda: (0, 0)),
        out_shape=jax.ShapeDtypeStruct((1, 10000), jnp.float32),
    )(x, W1, b1, W2, b2)


def _sc_msgpass_body(T_hbm, srcf_hbm, dstf_hbm, ee_hbm, out_hbm,
                     src_v, dst_v, srcg_v, dstg_v, srcgb_v, dstgb_v, ee_v,
                     ts_a, ts_b, td_a, td_b, msg_v, A_sh, sem):
    cid = lax.axis_index("c")
    sid = lax.axis_index("s")

    zero16 = jnp.zeros((16,), jnp.float32)

    def _zrow(r, carry):
        for j in range(AW // 16):
            msg_v[r, pl.ds(j * 16, 16)] = zero16
        return carry
    lax.fori_loop(0, CH, _zrow, 0)

    def _zcp(kk, carry):
        pltpu.sync_copy(msg_v, A_sh.at[pl.ds(sid * ROWS_PER_TILE + kk * CH, CH)])
        return carry
    lax.fori_loop(0, ROWS_PER_TILE // CH, _zcp, 0)

    iot = lax.iota(jnp.int32, 16)
    one0 = jnp.where(iot == 0, jnp.float32(1.0), jnp.float32(0.0))

    def _ones(r, carry):
        msg_v[r, pl.ds(HID, 16)] = one0
        return carry
    lax.fori_loop(0, CH, _ones, 0)

    plsc.subcore_barrier()

    base0 = cid * EPAD + sid * EPW
    goff = cid * NR

    def _chunk(ci, carry):
        base = base0 + ci * CH
        pltpu.sync_copy(srcf_hbm.at[pl.ds(base, CH)], src_v)
        pltpu.sync_copy(dstf_hbm.at[pl.ds(base, CH)], dst_v)

        def _adj(j, c2):
            sl = pl.ds(j * 16, 16)
            sg = src_v[sl] + goff
            dg = dst_v[sl] + goff
            srcg_v[sl] = sg
            dstg_v[sl] = dg
            srcgb_v[sl] = sg + NRTOT
            dstgb_v[sl] = dg + NRTOT
            return c2
        lax.fori_loop(0, CH // 16, _adj, 0)

        c0 = pltpu.async_copy(ee_hbm.at[pl.ds(base, CH)], ee_v, sem)
        c1 = pltpu.async_copy(T_hbm.at[srcg_v], ts_a, sem)
        c2 = pltpu.async_copy(T_hbm.at[dstg_v], td_a, sem)
        c3 = pltpu.async_copy(T_hbm.at[srcgb_v], ts_b, sem)
        c4 = pltpu.async_copy(T_hbm.at[dstgb_v], td_b, sem)
        c0.wait()
        c1.wait()
        c2.wait()
        c3.wait()
        c4.wait()

        def _row_d(r, c2):
            for j in range(HID // 16):
                sl = pl.ds(j * 16, 16)
                msg_v[r, sl] = jnp.maximum(ts_a[r, sl] + ee_v[r, sl] + td_b[r, sl], 0.0)
            return c2
        lax.fori_loop(0, CH, _row_d, 0)
        pltpu.sync_copy(msg_v, A_sh.at[dst_v], add=True)

        def _row_s(r, c2):
            for j in range(HID // 16):
                sl = pl.ds(j * 16, 16)
                msg_v[r, sl] = jnp.maximum(td_a[r, sl] + ee_v[r, sl] + ts_b[r, sl], 0.0)
            return c2
        lax.fori_loop(0, CH, _row_s, 0)
        pltpu.sync_copy(msg_v, A_sh.at[src_v], add=True)
        return carry
    lax.fori_loop(0, NCHUNK, _chunk, 0)

    plsc.subcore_barrier()

    def _wb(kk, carry):
        r0 = sid * ROWS_PER_TILE + kk * CH
        pltpu.sync_copy(A_sh.at[pl.ds(r0, CH)], msg_v)
        pltpu.sync_copy(msg_v, out_hbm.at[pl.ds(goff + r0, CH)])
        return carry
    lax.fori_loop(0, ROWS_PER_TILE // CH, _wb, 0)


@functools.cache
def _get_sc_msgpass():
    mesh = plsc.VectorSubcoreMesh(core_axis_name="c", subcore_axis_name="s",
                                  num_cores=2, num_subcores=16)
    return pl.kernel(
        _sc_msgpass_body,
        mesh=mesh,
        compiler_params=pltpu.CompilerParams(use_tc_tiling_on_sc=False),
        out_type=jax.ShapeDtypeStruct((NRTOT, AW), jnp.float32),
        scratch_types=[
            pltpu.VMEM((CH,), jnp.int32),
            pltpu.VMEM((CH,), jnp.int32),
            pltpu.VMEM((CH,), jnp.int32),
            pltpu.VMEM((CH,), jnp.int32),
            pltpu.VMEM((CH,), jnp.int32),
            pltpu.VMEM((CH,), jnp.int32),
            pltpu.VMEM((CH, HID), jnp.float32),
            pltpu.VMEM((CH, HID), jnp.float32),
            pltpu.VMEM((CH, HID), jnp.float32),
            pltpu.VMEM((CH, HID), jnp.float32),
            pltpu.VMEM((CH, HID), jnp.float32),
            pltpu.VMEM((CH, AW), jnp.float32),
            pltpu.VMEM_SHARED((NR, AW), jnp.float32),
            pltpu.SemaphoreType.DMA,
        ],
    )


def _sc_debug_emu(T, src_f, dst_f, Ee):
    As = []
    for sct in range(2):
        sl = slice(sct * EPAD, (sct + 1) * EPAD)
        srcl, dstl, ee = src_f[sl], dst_f[sl], Ee[sl]
        srcg, dstg = srcl + sct * NR, dstl + sct * NR
        rd = jax.nn.relu(T[srcg] + ee + T[dstg + NRTOT])
        rs = jax.nn.relu(T[dstg] + ee + T[srcg + NRTOT])
        ones = jnp.ones((EPAD, 1), jnp.float32)
        zpad = jnp.zeros((EPAD, AW - HID - 1), jnp.float32)
        rowd = jnp.concatenate([rd, ones, zpad], 1)
        rows = jnp.concatenate([rs, ones, zpad], 1)
        Acc = jnp.zeros((NR, AW), jnp.float32)
        Acc = Acc.at[dstl].add(rowd).at[srcl].add(rows)
        As.append(Acc)
    return jnp.concatenate(As, axis=0)


def kernel(params, node_features, edge_list, edge_features, timestamps, target_time):
    f32 = jnp.float32

    def r2(b):
        return b.reshape(1, -1)

    (nW1, nb1), (nW2, nb2) = params['node_enc']
    (eW1, eb1), (eW2, eb2) = params['edge_enc']
    (gW1, gb1), (gW2, gb2) = params['agg']
    msg = params['msg']
    W1a, W1c, Wb, bb, W2aug = [], [], [], [], []
    for lp in msg:
        (W1, b1), (W2, b2) = lp
        W1a.append(W1[0:HID])
        Wb.append(W1[HID:2 * HID])
        W1c.append(W1[2 * HID:3 * HID])
        bb.append(r2(b1))
        w2a = jnp.concatenate([W2, b2[None, :]], axis=0)
        W2aug.append(jnp.pad(w2a, ((0, AW - HID - 1), (0, 0))))

    nf_f = jnp.pad(node_features, ((0, 0), (0, NR - N), (0, 0))).reshape(NRTOT, NFD)
    src = edge_list[:, :, 0]
    dst = edge_list[:, :, 1]
    src_f = jnp.pad(src, ((0, 0), (0, EPAD - E)), constant_values=N).reshape(EF)
    dst_f = jnp.pad(dst, ((0, 0), (0, EPAD - E)), constant_values=N).reshape(EF)
    ef_f = jnp.pad(edge_features, ((0, 0), (0, EPAD - E), (0, 0))).reshape(EF, EFD)

    h_f = _node_enc(nf_f, nW1, r2(nb1), nW2, r2(nb2))
    Ee0, Ee1 = _edge_enc(ef_f, eW1, r2(eb1), eW2, r2(eb2),
                         Wb[0], bb[0], Wb[1], bb[1])
    Ee = (Ee0, Ee1)

    for li in range(2):
        G1, G3 = _tbuild(h_f, W1a[li], W1c[li])
        T = jnp.concatenate([G1, G3], axis=0)
        A = _get_sc_msgpass()(T, src_f, dst_f, Ee[li])
        h_f = _combine(h_f, A, W2aug[li], gW1, r2(gb1), gW2, r2(gb2))

    (ttW1, ttb1), (ttW2, ttb2) = params['t_time']
    (gtW1, gtb1), (gtW2, gtb2) = params['g_time']
    tf = params['tf']
    stk = lambda key: jnp.stack([lp[key] for lp in tf])
    stk2 = lambda key: jnp.stack([lp[key][None, :] for lp in tf])
    targs = [jnp.asarray(_PE),
             ttW1, r2(ttb1), ttW2, r2(ttb2),
             gtW1, r2(gtb1), gtW2, r2(gtb2),
             stk('Wqkv'), stk2('bqkv'), stk('Wo'), stk2('bo'),
             stk('W1'), stk2('b1'), stk('W2'), stk2('b2'),
             stk2('g1'), stk2('be1'), stk2('g2'), stk2('be2')]
    (cW1, cb1), (cW2, cb2) = params['ctx']
    (sW1, sb1), (sW2, sb2) = params['srcp']
    targs += [cW1, r2(cb1), cW2, r2(cb2), sW1, r2(sb1), sW2, r2(sb2)]
    ts2 = timestamps[:, None].astype(f32)
    tt2 = target_time.reshape(1, 1).astype(f32)
    ctx, src_logits2 = _tail(h_f, ts2, tt2, targs)
    src_logits = src_logits2[0]

    src_node = jax.random.categorical(jax.random.key(42), src_logits)
    src_emb = jax.lax.dynamic_slice(h_f, (NR + src_node, 0), (1, HID))
    (dW1, db1), (dW2, db2) = params['dstp']
    x_d = jnp.concatenate([ctx, src_emb], axis=1)
    dst_logits = _dst_mlp(x_d, dW1, r2(db1), dW2, r2(db2))[0]

    return src_logits, dst_logits, src_node


# speedup vs baseline: 1.9732x; 1.0612x over previous
import functools
import math

import jax
import jax.numpy as jnp
import numpy as np
from jax import lax
from jax.experimental import pallas as pl
from jax.experimental.pallas import tpu as pltpu
from jax.experimental.pallas import tpu_sc as plsc

HID = 64
NFD = 128
EFD = 16
N = 10000
L = 2
E = 160000
NR = 10240
NRTOT = 2 * NR
EPAD = 163840
EF = 2 * EPAD
CH = 128
NWORK = 32
EPW = EPAD // 16
NCHUNK = EPW // CH
ROWS_PER_TILE = NR // 16
AW = 80


def _pos_encoding(Lx, d):
    pos = np.arange(Lx)[:, None].astype(np.float32)
    div = np.exp(np.arange(0, d, 2).astype(np.float32) * -(math.log(10000.0) / d))
    pe = np.zeros((Lx, d), np.float32)
    pe[:, 0::2] = np.sin(pos * div)
    pe[:, 1::2] = np.cos(pos * div)
    return pe


_PE = _pos_encoding(L, HID)


def _whole(shape):
    return pl.BlockSpec(shape, lambda *_: tuple(0 for _ in shape))


def _mlp2_body(x, W1, b1, W2, b2):
    hcur = jnp.maximum(jnp.dot(x, W1, preferred_element_type=jnp.float32) + b1, 0.0)
    return jnp.dot(hcur, W2, preferred_element_type=jnp.float32) + b2


def _node_enc_k(nf_ref, W1_ref, b1_ref, W2_ref, b2_ref, out_ref):
    out_ref[...] = _mlp2_body(nf_ref[...], W1_ref[...], b1_ref[...],
                              W2_ref[...], b2_ref[...])


def _node_enc(nf_f, W1, b1, W2, b2):
    B = 2048
    grid = (NRTOT // B,)
    return pl.pallas_call(
        _node_enc_k,
        grid=grid,
        in_specs=[pl.BlockSpec((B, NFD), lambda i: (i, 0)),
                  _whole(W1.shape), _whole(b1.shape),
                  _whole(W2.shape), _whole(b2.shape)],
        out_specs=pl.BlockSpec((B, HID), lambda i: (i, 0)),
        out_shape=jax.ShapeDtypeStruct((NRTOT, HID), jnp.float32),
    )(nf_f, W1, b1, W2, b2)


def _edge_enc_k(ef_ref, W1_ref, b1_ref, W2_ref, b2_ref,
                Wb0_ref, bb0_ref, Wb1_ref, bb1_ref, e0_ref, e1_ref):
    e = _mlp2_body(ef_ref[...], W1_ref[...], b1_ref[...], W2_ref[...], b2_ref[...])
    e0_ref[...] = jnp.dot(e, Wb0_ref[...], preferred_element_type=jnp.float32) + bb0_ref[...]
    e1_ref[...] = jnp.dot(e, Wb1_ref[...], preferred_element_type=jnp.float32) + bb1_ref[...]


def _edge_enc(ef_f, W1, b1, W2, b2, Wb0, bb0, Wb1, bb1):
    B = 4096
    grid = (EF // B,)
    return pl.pallas_call(
        _edge_enc_k,
        grid=grid,
        in_specs=[pl.BlockSpec((B, EFD), lambda i: (i, 0)),
                  _whole(W1.shape), _whole(b1.shape),
                  _whole(W2.shape), _whole(b2.shape),
                  _whole(Wb0.shape), _whole(bb0.shape),
                  _whole(Wb1.shape), _whole(bb1.shape)],
        out_specs=[pl.BlockSpec((B, HID), lambda i: (i, 0)),
                   pl.BlockSpec((B, HID), lambda i: (i, 0))],
        out_shape=[jax.ShapeDtypeStruct((EF, HID), jnp.float32),
                   jax.ShapeDtypeStruct((EF, HID), jnp.float32)],
    )(ef_f, W1, b1, W2, b2, Wb0, bb0, Wb1, bb1)


def _tbuild_k(h_ref, Wa_ref, Wc_ref, ga_ref, gc_ref):
    h = h_ref[...]
    ga_ref[...] = jnp.dot(h, Wa_ref[...], preferred_element_type=jnp.float32)
    gc_ref[...] = jnp.dot(h, Wc_ref[...], preferred_element_type=jnp.float32)


def _tbuild(h_f, Wa, Wc):
    B = 2048
    grid = (NRTOT // B,)
    return pl.pallas_call(
        _tbuild_k,
        grid=grid,
        in_specs=[pl.BlockSpec((B, HID), lambda i: (i, 0)),
                  _whole(Wa.shape), _whole(Wc.shape)],
        out_specs=[pl.BlockSpec((B, HID), lambda i: (i, 0)),
                   pl.BlockSpec((B, HID), lambda i: (i, 0))],
        out_shape=[jax.ShapeDtypeStruct((NRTOT, HID), jnp.float32),
                   jax.ShapeDtypeStruct((NRTOT, HID), jnp.float32)],
    )(h_f, Wa, Wc)


def _combine_k(h_ref, A_ref, W2a_ref, Wg1_ref, bg1_ref, Wg2_ref, bg2_ref, out_ref):
    msg = jnp.dot(A_ref[...], W2a_ref[...], preferred_element_type=jnp.float32)
    new_h = h_ref[...] + msg
    out_ref[...] = _mlp2_body(new_h, Wg1_ref[...], bg1_ref[...],
                              Wg2_ref[...], bg2_ref[...])


def _combine(h_f, A, W2aug, Wg1, bg1, Wg2, bg2):
    B = 2048
    grid = (NRTOT // B,)
    return pl.pallas_call(
        _combine_k,
        grid=grid,
        in_specs=[pl.BlockSpec((B, HID), lambda i: (i, 0)),
                  pl.BlockSpec((B, AW), lambda i: (i, 0)),
                  _whole(W2aug.shape),
                  _whole(Wg1.shape), _whole(bg1.shape),
                  _whole(Wg2.shape), _whole(bg2.shape)],
        out_specs=pl.BlockSpec((B, HID), lambda i: (i, 0)),
        out_shape=jax.ShapeDtypeStruct((NRTOT, HID), jnp.float32),
    )(h_f, A, W2aug, Wg1, bg1, Wg2, bg2)


def _ln(x, g, b):
    m = jnp.mean(x, axis=-1, keepdims=True)
    v = jnp.mean((x - m) ** 2, axis=-1, keepdims=True)
    return (x - m) / jnp.sqrt(v + 1e-5) * g + b


def _tail_k(h_ref, ts_ref, tt_ref, pe_ref,
            ttW1_ref, ttb1_ref, ttW2_ref, ttb2_ref,
            gtW1_ref, gtb1_ref, gtW2_ref, gtb2_ref,
            Wqkv_ref, bqkv_ref, Wo_ref, bo_ref,
            Wf1_ref, bf1_ref, Wf2_ref, bf2_ref,
            g1_ref, be1_ref, g2_ref, be2_ref,
            cW1_ref, cb1_ref, cW2_ref, cb2_ref,
            sW1_ref, sb1_ref, sW2_ref, sb2_ref,
            ctx_ref, logits_ref):
    s0 = jnp.mean(h_ref[0:N, :], axis=0)
    s1 = jnp.mean(h_ref[NR:NR + N, :], axis=0)
    ts_emb = jnp.concatenate([s0[None, :], s1[None, :]], axis=0)
    time_emb = _mlp2_body_bc(ts_ref[...], ttW1_ref[...], ttb1_ref[...],
                             ttW2_ref[...], ttb2_ref[...])
    x = ts_emb + time_emb + pe_ref[...]
    inv = 1.0 / math.sqrt(HID // 8)
    for l in range(6):
        qkv = jnp.dot(x, Wqkv_ref[l], preferred_element_type=jnp.float32) + bqkv_ref[l]
        q = qkv[:, 0:HID]
        k = qkv[:, HID:2 * HID]
        v = qkv[:, 2 * HID:3 * HID]
        outs = []
        for hh in range(8):
            sl = slice(hh * 8, hh * 8 + 8)
            qh, kh, vh = q[:, sl], k[:, sl], v[:, sl]
            s00 = jnp.sum(qh[0] * kh[0]) * inv
            s01 = jnp.sum(qh[0] * kh[1]) * inv
            s10 = jnp.sum(qh[1] * kh[0]) * inv
            s11 = jnp.sum(qh[1] * kh[1]) * inv
            m0 = jnp.maximum(s00, s01)
            e00 = jnp.exp(s00 - m0)
            e01 = jnp.exp(s01 - m0)
            a00 = e00 / (e00 + e01)
            a01 = e01 / (e00 + e01)
            m1 = jnp.maximum(s10, s11)
            e10 = jnp.exp(s10 - m1)
            e11 = jnp.exp(s11 - m1)
            a10 = e10 / (e10 + e11)
            a11 = e11 / (e10 + e11)
            o0 = a00 * vh[0] + a01 * vh[1]
            o1 = a10 * vh[0] + a11 * vh[1]
            outs.append(jnp.concatenate([o0[None, :], o1[None, :]], axis=0))
        o = jnp.concatenate(outs, axis=1)
        o = jnp.dot(o, Wo_ref[l], preferred_element_type=jnp.float32) + bo_ref[l]
        x = _ln(x + o, g1_ref[l], be1_ref[l])
        ff = jnp.maximum(jnp.dot(x, Wf1_ref[l], preferred_element_type=jnp.float32) + bf1_ref[l], 0.0)
        ff = jnp.dot(ff, Wf2_ref[l], preferred_element_type=jnp.float32) + bf2_ref[l]
        x = _ln(x + ff, g2_ref[l], be2_ref[l])
    seq_ctx = x[1]
    t_ctx = _mlp2_body_bc(tt_ref[...], gtW1_ref[...], gtb1_ref[...],
                          gtW2_ref[...], gtb2_ref[...])
    ctx_in = jnp.concatenate([s1[None, :], seq_ctx[None, :], t_ctx], axis=1)
    ctx = _mlp2_body(ctx_in, cW1_ref[...], cb1_ref[...], cW2_ref[...], cb2_ref[...])
    ctx_ref[...] = ctx
    logits_ref[...] = _mlp2_body(ctx, sW1_ref[...], sb1_ref[...],
                                 sW2_ref[...], sb2_ref[...])


def _mlp2_body_bc(x1, W1row, b1, W2, b2):
    hcur = jnp.maximum(x1 * W1row + b1, 0.0)
    return jnp.dot(hcur, W2, preferred_element_type=jnp.float32) + b2


def _tail(h_f, ts, tt, args):
    in_specs = [_whole(h_f.shape), _whole(ts.shape), _whole(tt.shape)]
    ops = [h_f, ts, tt]
    for a in args:
        in_specs.append(_whole(a.shape))
        ops.append(a)
    return pl.pallas_call(
        _tail_k,
        in_specs=in_specs,
        out_specs=[pl.BlockSpec((1, HID), lambda: (0, 0)),
                   pl.BlockSpec((1, 10000), lambda: (0, 0))],
        out_shape=[jax.ShapeDtypeStruct((1, HID), jnp.float32),
                   jax.ShapeDtypeStruct((1, 10000), jnp.float32)],
    )(*ops)


def _dst_k(x_ref, W1_ref, b1_ref, W2_ref, b2_ref, out_ref):
    out_ref[...] = _mlp2_body(x_ref[...], W1_ref[...], b1_ref[...],
                              W2_ref[...], b2_ref[...])


def _dst_mlp(x, W1, b1, W2, b2):
    return pl.pallas_call(
        _dst_k,
        in_specs=[_whole(x.shape), _whole(W1.shape), _whole(b1.shape),
                  _whole(W2.shape), _whole(b2.shape)],
        out_specs=pl.BlockSpec((1, 10000), lambda: (0, 0)),
        out_shape=jax.ShapeDtypeStruct((1, 10000), jnp.float32),
    )(x, W1, b1, W2, b2)


def _sc_msgpass_body(T_hbm, srcf_hbm, dstf_hbm, ee_hbm, out_hbm,
                     src_v, dst_v, srcg_v, dstg_v, srcgb_v, dstgb_v, ee_v,
                     ts_a, ts_b, td_a, td_b, msg_v, A_sh, sem):
    cid = lax.axis_index("c")
    sid = lax.axis_index("s")

    zero16 = jnp.zeros((16,), jnp.float32)

    def _zrow(r, carry):
        for j in range(AW // 16):
            msg_v[r, pl.ds(j * 16, 16)] = zero16
        return carry
    lax.fori_loop(0, CH, _zrow, 0)

    def _zcp(kk, carry):
        pltpu.sync_copy(msg_v, A_sh.at[pl.ds(sid * ROWS_PER_TILE + kk * CH, CH)])
        return carry
    lax.fori_loop(0, ROWS_PER_TILE // CH, _zcp, 0)

    iot = lax.iota(jnp.int32, 16)
    one0 = jnp.where(iot == 0, jnp.float32(1.0), jnp.float32(0.0))

    def _ones(r, carry):
        msg_v[r, pl.ds(HID, 16)] = one0
        return carry
    lax.fori_loop(0, CH, _ones, 0)

    plsc.subcore_barrier()

    base0 = cid * EPAD + sid * EPW
    goff = cid * NR

    def _chunk(ci, carry):
        base = base0 + ci * CH
        pltpu.sync_copy(srcf_hbm.at[pl.ds(base, CH)], src_v)
        pltpu.sync_copy(dstf_hbm.at[pl.ds(base, CH)], dst_v)

        def _adj(j, c2):
            sl = pl.ds(j * 16, 16)
            sg = src_v[sl] + goff
            dg = dst_v[sl] + goff
            srcg_v[sl] = sg
            dstg_v[sl] = dg
            srcgb_v[sl] = sg + NRTOT
            dstgb_v[sl] = dg + NRTOT
            return c2
        lax.fori_loop(0, CH // 16, _adj, 0)

        c0 = pltpu.async_copy(ee_hbm.at[pl.ds(base, CH)], ee_v, sem)
        c1 = pltpu.async_copy(T_hbm.at[srcg_v], ts_a, sem)
        c2 = pltpu.async_copy(T_hbm.at[dstg_v], td_a, sem)
        c3 = pltpu.async_copy(T_hbm.at[srcgb_v], ts_b, sem)
        c4 = pltpu.async_copy(T_hbm.at[dstgb_v], td_b, sem)
        c0.wait()
        c1.wait()
        c4.wait()

        def _row_d(r, cc):
            for j in range(HID // 16):
                sl = pl.ds(j * 16, 16)
                msg_v[r, sl] = jnp.maximum(ts_a[r, sl] + ee_v[r, sl] + td_b[r, sl], 0.0)
            return cc
        lax.fori_loop(0, CH, _row_d, 0, unroll=4)
        pltpu.sync_copy(msg_v, A_sh.at[dst_v], add=True)

        c2.wait()
        c3.wait()

        def _row_s(r, cc):
            for j in range(HID // 16):
                sl = pl.ds(j * 16, 16)
                msg_v[r, sl] = jnp.maximum(td_a[r, sl] + ee_v[r, sl] + ts_b[r, sl], 0.0)
            return cc
        lax.fori_loop(0, CH, _row_s, 0, unroll=4)
        pltpu.sync_copy(msg_v, A_sh.at[src_v], add=True)
        return carry
    lax.fori_loop(0, NCHUNK, _chunk, 0)

    plsc.subcore_barrier()

    def _wb(kk, carry):
        r0 = sid * ROWS_PER_TILE + kk * CH
        pltpu.sync_copy(A_sh.at[pl.ds(r0, CH)], msg_v)
        pltpu.sync_copy(msg_v, out_hbm.at[pl.ds(goff + r0, CH)])
        return carry
    lax.fori_loop(0, ROWS_PER_TILE // CH, _wb, 0)


@functools.cache
def _get_sc_msgpass():
    mesh = plsc.VectorSubcoreMesh(core_axis_name="c", subcore_axis_name="s",
                                  num_cores=2, num_subcores=16)
    return pl.kernel(
        _sc_msgpass_body,
        mesh=mesh,
        compiler_params=pltpu.CompilerParams(use_tc_tiling_on_sc=False),
        out_type=jax.ShapeDtypeStruct((NRTOT, AW), jnp.float32),
        scratch_types=[
            pltpu.VMEM((CH,), jnp.int32),
            pltpu.VMEM((CH,), jnp.int32),
            pltpu.VMEM((CH,), jnp.int32),
            pltpu.VMEM((CH,), jnp.int32),
            pltpu.VMEM((CH,), jnp.int32),
            pltpu.VMEM((CH,), jnp.int32),
            pltpu.VMEM((CH, HID), jnp.float32),
            pltpu.VMEM((CH, HID), jnp.float32),
            pltpu.VMEM((CH, HID), jnp.float32),
            pltpu.VMEM((CH, HID), jnp.float32),
            pltpu.VMEM((CH, HID), jnp.float32),
            pltpu.VMEM((CH, AW), jnp.float32),
            pltpu.VMEM_SHARED((NR, AW), jnp.float32),
            pltpu.SemaphoreType.DMA,
        ],
    )


def _sc_debug_emu(T, src_f, dst_f, Ee):
    As = []
    for sct in range(2):
        sl = slice(sct * EPAD, (sct + 1) * EPAD)
        srcl, dstl, ee = src_f[sl], dst_f[sl], Ee[sl]
        srcg, dstg = srcl + sct * NR, dstl + sct * NR
        rd = jax.nn.relu(T[srcg] + ee + T[dstg + NRTOT])
        rs = jax.nn.relu(T[dstg] + ee + T[srcg + NRTOT])
        ones = jnp.ones((EPAD, 1), jnp.float32)
        zpad = jnp.zeros((EPAD, AW - HID - 1), jnp.float32)
        rowd = jnp.concatenate([rd, ones, zpad], 1)
        rows = jnp.concatenate([rs, ones, zpad], 1)
        Acc = jnp.zeros((NR, AW), jnp.float32)
        Acc = Acc.at[dstl].add(rowd).at[srcl].add(rows)
        As.append(Acc)
    return jnp.concatenate(As, axis=0)


def kernel(params, node_features, edge_list, edge_features, timestamps, target_time):
    f32 = jnp.float32

    def r2(b):
        return b.reshape(1, -1)

    (nW1, nb1), (nW2, nb2) = params['node_enc']
    (eW1, eb1), (eW2, eb2) = params['edge_enc']
    (gW1, gb1), (gW2, gb2) = params['agg']
    msg = params['msg']
    W1a, W1c, Wb, bb, W2aug = [], [], [], [], []
    for lp in msg:
        (W1, b1), (W2, b2) = lp
        W1a.append(W1[0:HID])
        Wb.append(W1[HID:2 * HID])
        W1c.append(W1[2 * HID:3 * HID])
        bb.append(r2(b1))
        w2a = jnp.concatenate([W2, b2[None, :]], axis=0)
        W2aug.append(jnp.pad(w2a, ((0, AW - HID - 1), (0, 0))))

    nf_f = jnp.pad(node_features, ((0, 0), (0, NR - N), (0, 0))).reshape(NRTOT, NFD)
    src = edge_list[:, :, 0]
    dst = edge_list[:, :, 1]
    src_f = jnp.pad(src, ((0, 0), (0, EPAD - E)), constant_values=N).reshape(EF)
    dst_f = jnp.pad(dst, ((0, 0), (0, EPAD - E)), constant_values=N).reshape(EF)
    ef_f = jnp.pad(edge_features, ((0, 0), (0, EPAD - E), (0, 0))).reshape(EF, EFD)

    h_f = _node_enc(nf_f, nW1, r2(nb1), nW2, r2(nb2))
    Ee0, Ee1 = _edge_enc(ef_f, eW1, r2(eb1), eW2, r2(eb2),
                         Wb[0], bb[0], Wb[1], bb[1])
    Ee = (Ee0, Ee1)

    for li in range(2):
        G1, G3 = _tbuild(h_f, W1a[li], W1c[li])
        T = jnp.concatenate([G1, G3], axis=0)
        A = _get_sc_msgpass()(T, src_f, dst_f, Ee[li])
        h_f = _combine(h_f, A, W2aug[li], gW1, r2(gb1), gW2, r2(gb2))

    (ttW1, ttb1), (ttW2, ttb2) = params['t_time']
    (gtW1, gtb1), (gtW2, gtb2) = params['g_time']
    tf = params['tf']
    stk = lambda key: jnp.stack([lp[key] for lp in tf])
    stk2 = lambda key: jnp.stack([lp[key][None, :] for lp in tf])
    targs = [jnp.asarray(_PE),
             ttW1, r2(ttb1), ttW2, r2(ttb2),
             gtW1, r2(gtb1), gtW2, r2(gtb2),
             stk('Wqkv'), stk2('bqkv'), stk('Wo'), stk2('bo'),
             stk('W1'), stk2('b1'), stk('W2'), stk2('b2'),
             stk2('g1'), stk2('be1'), stk2('g2'), stk2('be2')]
    (cW1, cb1), (cW2, cb2) = params['ctx']
    (sW1, sb1), (sW2, sb2) = params['srcp']
    targs += [cW1, r2(cb1), cW2, r2(cb2), sW1, r2(sb1), sW2, r2(sb2)]
    ts2 = timestamps[:, None].astype(f32)
    tt2 = target_time.reshape(1, 1).astype(f32)
    ctx, src_logits2 = _tail(h_f, ts2, tt2, targs)
    src_logits = src_logits2[0]

    src_node = jax.random.categorical(jax.random.key(42), src_logits)
    src_emb = jax.lax.dynamic_slice(h_f, (NR + src_node, 0), (1, HID))
    (dW1, db1), (dW2, db2) = params['dstp']
    x_d = jnp.concatenate([ctx, src_emb], axis=1)
    dst_logits = _dst_mlp(x_d, dW1, r2(db1), dW2, r2(db2))[0]

    return src_logits, dst_logits, src_node
